# baseline (speedup 1.0000x reference)
import functools

import jax
import jax.numpy as jnp
from jax import lax
from jax.experimental import pallas as pl
from jax.experimental.pallas import tpu as pltpu
from jax.experimental.pallas import tpu_sc as plsc

BATCH = 16384
EMB = 32
L = 16

_info = plsc.get_sparse_core_info()
NC = _info.num_cores
NS = _info.num_subcores
NW = NC * NS
BPW = BATCH // NW


def _sc_body(uid_hbm, iid_hbm, ut_hbm, it_hbm, wb_hbm, out_hbm,
             idx_u, idx_i, rows_u, rows_i, wv, out_v, sem_u, sem_i):
    wid = lax.axis_index("s") * NC + lax.axis_index("c")
    base = wid * BPW

    pltpu.sync_copy(wb_hbm, wv)
    pltpu.sync_copy(uid_hbm.at[pl.ds(base, BPW)], idx_u)
    pltpu.sync_copy(iid_hbm.at[pl.ds(base, BPW)], idx_i)

    cu = pltpu.async_copy(ut_hbm.at[idx_u], rows_u, sem_u)
    ci = pltpu.async_copy(it_hbm.at[idx_i], rows_i, sem_i)
    cu.wait()
    ci.wait()

    wu0 = wv[pl.ds(0, L)]
    wu1 = wv[pl.ds(L, L)]
    wi0 = wv[pl.ds(2 * L, L)]
    wi1 = wv[pl.ds(3 * L, L)]
    bias_vec = wv[pl.ds(4 * L, L)]

    def group(g, carry):
        r0 = g * L
        out_v[pl.ds(r0, L)] = bias_vec
        for j in range(L):
            r = r0 + j
            a = (rows_u[r, pl.ds(0, L)] * wu0 + rows_u[r, pl.ds(L, L)] * wu1
                 + rows_i[r, pl.ds(0, L)] * wi0
                 + rows_i[r, pl.ds(L, L)] * wi1)
            plsc.addupdate_scatter(out_v, [jnp.full((L,), r, jnp.int32)], a)
        return carry

    lax.fori_loop(0, BPW // L, group, 0)
    pltpu.sync_copy(out_v, out_hbm.at[pl.ds(base, BPW)])


@jax.jit
def _run(user_ids, item_ids, user_table, item_table, wb):
    mesh = plsc.VectorSubcoreMesh(core_axis_name="c", subcore_axis_name="s")
    k = functools.partial(
        pl.kernel,
        mesh=mesh,
        compiler_params=pltpu.CompilerParams(
            needs_layout_passes=False, use_tc_tiling_on_sc=False),
        out_type=jax.ShapeDtypeStruct((BATCH,), jnp.float32),
        scratch_types=[
            pltpu.VMEM((BPW,), jnp.int32),
            pltpu.VMEM((BPW,), jnp.int32),
            pltpu.VMEM((BPW, EMB), jnp.float32),
            pltpu.VMEM((BPW, EMB), jnp.float32),
            pltpu.VMEM((5 * L,), jnp.float32),
            pltpu.VMEM((BPW,), jnp.float32),
            pltpu.SemaphoreType.DMA,
            pltpu.SemaphoreType.DMA,
        ],
    )(_sc_body)
    return k(user_ids, item_ids, user_table, item_table, wb)


def kernel(user_ids, item_ids, user_table, item_table, fc_w, fc_b):
    wb = jnp.concatenate(
        [fc_w.reshape(-1), jnp.broadcast_to(fc_b.reshape(-1), (L,))])
    return _run(user_ids.astype(jnp.int32), item_ids.astype(jnp.int32),
                user_table, item_table, wb)


# speedup vs baseline: 3.7624x; 3.7624x over previous
import functools

import jax
import jax.numpy as jnp
from jax import lax
from jax.experimental import pallas as pl
from jax.experimental.pallas import tpu as pltpu
from jax.experimental.pallas import tpu_sc as plsc

BATCH = 16384
EMB = 32
L = 16

_info = plsc.get_sparse_core_info()
NC = _info.num_cores
NS = _info.num_subcores
NW = NC * NS
BPW = BATCH // NW

TC_BLK = 16384


def _col_dot(table_t, w_col):
    n = table_t.shape[1]

    def body(t_ref, w_ref, o_ref):
        o_ref[...] = jnp.sum(t_ref[...] * w_ref[...], axis=0, keepdims=True)

    return pl.pallas_call(
        body,
        grid=(pl.cdiv(n, TC_BLK),),
        in_specs=[
            pl.BlockSpec((EMB, TC_BLK), lambda i: (0, i)),
            pl.BlockSpec((EMB, 1), lambda i: (0, 0)),
        ],
        out_specs=pl.BlockSpec((1, TC_BLK), lambda i: (0, i)),
        out_shape=jax.ShapeDtypeStruct((1, n), jnp.float32),
    )(table_t, w_col)


def _sc_body(uid_hbm, iid_hbm, su_hbm, si_hbm, wb_hbm, out_hbm,
             idx_u, idx_i, g_u, g_i, out_v, wv, sem_u, sem_i):
    wid = lax.axis_index("s") * NC + lax.axis_index("c")
    base = wid * BPW

    pltpu.sync_copy(wb_hbm, wv)
    pltpu.sync_copy(uid_hbm.at[pl.ds(base, BPW)], idx_u)
    pltpu.sync_copy(iid_hbm.at[pl.ds(base, BPW)], idx_i)

    cu = pltpu.async_copy(su_hbm.at[idx_u], g_u, sem_u)
    ci = pltpu.async_copy(si_hbm.at[idx_i], g_i, sem_i)
    cu.wait()
    ci.wait()

    bias_vec = wv[pl.ds(0, L)]

    def group(g, carry):
        r0 = g * L
        out_v[pl.ds(r0, L)] = (
            g_u[pl.ds(r0, L)] + g_i[pl.ds(r0, L)] + bias_vec)
        return carry

    lax.fori_loop(0, BPW // L, group, 0)
    pltpu.sync_copy(out_v, out_hbm.at[pl.ds(base, BPW)])


@jax.jit
def _run(user_ids, item_ids, user_table, item_table, fc_w, fc_b):
    w = fc_w.reshape(-1)
    wu_col = w[:EMB].reshape(EMB, 1)
    wi_col = w[EMB:].reshape(EMB, 1)
    su = _col_dot(user_table.T, wu_col).reshape(-1)
    si = _col_dot(item_table.T, wi_col).reshape(-1)
    wb = jnp.broadcast_to(fc_b.reshape(-1), (L,))

    mesh = plsc.VectorSubcoreMesh(core_axis_name="c", subcore_axis_name="s")
    k = functools.partial(
        pl.kernel,
        mesh=mesh,
        compiler_params=pltpu.CompilerParams(
            needs_layout_passes=False, use_tc_tiling_on_sc=False),
        out_type=jax.ShapeDtypeStruct((BATCH,), jnp.float32),
        scratch_types=[
            pltpu.VMEM((BPW,), jnp.int32),
            pltpu.VMEM((BPW,), jnp.int32),
            pltpu.VMEM((BPW,), jnp.float32),
            pltpu.VMEM((BPW,), jnp.float32),
            pltpu.VMEM((BPW,), jnp.float32),
            pltpu.VMEM((L,), jnp.float32),
            pltpu.SemaphoreType.DMA,
            pltpu.SemaphoreType.DMA,
        ],
    )(_sc_body)
    return k(user_ids, item_ids, su, si, wb)


def kernel(user_ids, item_ids, user_table, item_table, fc_w, fc_b):
    return _run(user_ids.astype(jnp.int32), item_ids.astype(jnp.int32),
                user_table, item_table, fc_w, fc_b)


# speedup vs baseline: 4.5388x; 1.2064x over previous
import functools

import jax
import jax.numpy as jnp
from jax import lax
from jax.experimental import pallas as pl
from jax.experimental.pallas import tpu as pltpu
from jax.experimental.pallas import tpu_sc as plsc

BATCH = 16384
EMB = 32
L = 16

_info = plsc.get_sparse_core_info()
NC = _info.num_cores
NS = _info.num_subcores
NW = NC * NS
BPW = BATCH // NW

TC_BLK = 65536


def _col_dot(table_t, w_col):
    n = table_t.shape[1]

    def body(t_ref, w_ref, o_ref):
        o_ref[...] = jnp.sum(t_ref[...] * w_ref[...], axis=0, keepdims=True)

    return pl.pallas_call(
        body,
        grid=(pl.cdiv(n, TC_BLK),),
        in_specs=[
            pl.BlockSpec((EMB, TC_BLK), lambda i: (0, i)),
            pl.BlockSpec((EMB, 1), lambda i: (0, 0)),
        ],
        out_specs=pl.BlockSpec((1, TC_BLK), lambda i: (0, i)),
        out_shape=jax.ShapeDtypeStruct((1, n), jnp.float32),
    )(table_t, w_col)


def _sc_body(uid_hbm, iid_hbm, su_hbm, si_hbm, wb_hbm, out_hbm,
             idx_u, idx_i, g_u, g_i, out_v, wv, sem_u, sem_i):
    wid = lax.axis_index("s") * NC + lax.axis_index("c")
    base = wid * BPW

    pltpu.sync_copy(wb_hbm, wv)
    pltpu.sync_copy(uid_hbm.at[pl.ds(base, BPW)], idx_u)
    pltpu.sync_copy(iid_hbm.at[pl.ds(base, BPW)], idx_i)

    cu = pltpu.async_copy(su_hbm.at[idx_u], g_u, sem_u)
    ci = pltpu.async_copy(si_hbm.at[idx_i], g_i, sem_i)
    cu.wait()
    ci.wait()

    bias_vec = wv[pl.ds(0, L)]

    def group(g, carry):
        r0 = g * L
        out_v[pl.ds(r0, L)] = (
            g_u[pl.ds(r0, L)] + g_i[pl.ds(r0, L)] + bias_vec)
        return carry

    lax.fori_loop(0, BPW // L, group, 0)
    pltpu.sync_copy(out_v, out_hbm.at[pl.ds(base, BPW)])


@jax.jit
def _run(user_ids, item_ids, user_table, item_table, fc_w, fc_b):
    w = fc_w.reshape(-1)
    wu_col = w[:EMB].reshape(EMB, 1)
    wi_col = w[EMB:].reshape(EMB, 1)
    su = _col_dot(user_table.T, wu_col).reshape(-1)
    si = _col_dot(item_table.T, wi_col).reshape(-1)
    wb = jnp.broadcast_to(fc_b.reshape(-1), (L,))

    mesh = plsc.VectorSubcoreMesh(core_axis_name="c", subcore_axis_name="s")
    k = functools.partial(
        pl.kernel,
        mesh=mesh,
        compiler_params=pltpu.CompilerParams(
            needs_layout_passes=False, use_tc_tiling_on_sc=False),
        out_type=jax.ShapeDtypeStruct((BATCH,), jnp.float32),
        scratch_types=[
            pltpu.VMEM((BPW,), jnp.int32),
            pltpu.VMEM((BPW,), jnp.int32),
            pltpu.VMEM((BPW,), jnp.float32),
            pltpu.VMEM((BPW,), jnp.float32),
            pltpu.VMEM((BPW,), jnp.float32),
            pltpu.VMEM((L,), jnp.float32),
            pltpu.SemaphoreType.DMA,
            pltpu.SemaphoreType.DMA,
        ],
    )(_sc_body)
    return k(user_ids, item_ids, su, si, wb)


def kernel(user_ids, item_ids, user_table, item_table, fc_w, fc_b):
    return _run(user_ids.astype(jnp.int32), item_ids.astype(jnp.int32),
                user_table, item_table, fc_w, fc_b)


# speedup vs baseline: 7.2967x; 1.6076x over previous
import functools

import jax
import jax.numpy as jnp
from jax import lax
from jax.experimental import pallas as pl
from jax.experimental.pallas import tpu as pltpu
from jax.experimental.pallas import tpu_sc as plsc

BATCH = 16384
EMB = 32
L = 16

_info = plsc.get_sparse_core_info()
NC = _info.num_cores
NS = _info.num_subcores
NW = NC * NS
BPW = BATCH // NW

TC_BLK = 131072


def _col_dot(table_t, w_col):
    n = table_t.shape[1]

    def body(t_ref, w_ref, o_ref):
        o_ref[...] = jnp.sum(t_ref[...] * w_ref[...], axis=0)

    return pl.pallas_call(
        body,
        grid=(pl.cdiv(n, TC_BLK),),
        in_specs=[
            pl.BlockSpec((EMB, TC_BLK), lambda i: (0, i)),
            pl.BlockSpec((EMB, 1), lambda i: (0, 0)),
        ],
        out_specs=pl.BlockSpec((TC_BLK,), lambda i: (i,)),
        out_shape=jax.ShapeDtypeStruct((n,), jnp.float32),
    )(table_t, w_col)


def _sc_body(uid_hbm, iid_hbm, su_hbm, si_hbm, wb_hbm, out_hbm,
             idx_u, idx_i, g_u, g_i, out_v, wv, sem_u, sem_i):
    wid = lax.axis_index("s") * NC + lax.axis_index("c")
    base = wid * BPW

    pltpu.sync_copy(wb_hbm, wv)
    pltpu.sync_copy(uid_hbm.at[pl.ds(base, BPW)], idx_u)
    pltpu.sync_copy(iid_hbm.at[pl.ds(base, BPW)], idx_i)

    cu = pltpu.async_copy(su_hbm.at[idx_u], g_u, sem_u)
    ci = pltpu.async_copy(si_hbm.at[idx_i], g_i, sem_i)
    cu.wait()
    ci.wait()

    bias_vec = wv[pl.ds(0, L)]

    def group(g, carry):
        r0 = g * L
        out_v[pl.ds(r0, L)] = (
            g_u[pl.ds(r0, L)] + g_i[pl.ds(r0, L)] + bias_vec)
        return carry

    lax.fori_loop(0, BPW // L, group, 0)
    pltpu.sync_copy(out_v, out_hbm.at[pl.ds(base, BPW)])


@jax.jit
def _run(user_ids, item_ids, user_table, item_table, fc_w, fc_b):
    w = fc_w.reshape(-1)
    wu_col = w[:EMB].reshape(EMB, 1)
    wi_col = w[EMB:].reshape(EMB, 1)
    su = _col_dot(user_table.T, wu_col)
    si = _col_dot(item_table.T, wi_col)
    wb = jnp.broadcast_to(fc_b.reshape(-1), (L,))

    mesh = plsc.VectorSubcoreMesh(core_axis_name="c", subcore_axis_name="s")
    k = functools.partial(
        pl.kernel,
        mesh=mesh,
        compiler_params=pltpu.CompilerParams(
            needs_layout_passes=False, use_tc_tiling_on_sc=False),
        out_type=jax.ShapeDtypeStruct((BATCH,), jnp.float32),
        scratch_types=[
            pltpu.VMEM((BPW,), jnp.int32),
            pltpu.VMEM((BPW,), jnp.int32),
            pltpu.VMEM((BPW,), jnp.float32),
            pltpu.VMEM((BPW,), jnp.float32),
            pltpu.VMEM((BPW,), jnp.float32),
            pltpu.VMEM((L,), jnp.float32),
            pltpu.SemaphoreType.DMA,
            pltpu.SemaphoreType.DMA,
        ],
    )(_sc_body)
    return k(user_ids, item_ids, su, si, wb)


def kernel(user_ids, item_ids, user_table, item_table, fc_w, fc_b):
    return _run(user_ids.astype(jnp.int32), item_ids.astype(jnp.int32),
                user_table, item_table, fc_w, fc_b)


# speedup vs baseline: 7.4877x; 1.0262x over previous
import functools

import jax
import jax.numpy as jnp
from jax import lax
from jax.experimental import pallas as pl
from jax.experimental.pallas import tpu as pltpu
from jax.experimental.pallas import tpu_sc as plsc

BATCH = 16384
EMB = 32
L = 16

_info = plsc.get_sparse_core_info()
NC = _info.num_cores
NS = _info.num_subcores
NW = NC * NS
BPW = BATCH // NW

TC_BLK = 131072


def _col_dot(table_t, w_col):
    n = table_t.shape[1]

    def body(t_ref, w_ref, o_ref):
        prod = jax.lax.dot_general(
            w_ref[...], t_ref[...],
            (((0,), (0,)), ((), ())),
            preferred_element_type=jnp.float32)
        o_ref[...] = prod.reshape(-1)

    return pl.pallas_call(
        body,
        grid=(pl.cdiv(n, TC_BLK),),
        in_specs=[
            pl.BlockSpec((EMB, TC_BLK), lambda i: (0, i)),
            pl.BlockSpec((EMB, 1), lambda i: (0, 0)),
        ],
        out_specs=pl.BlockSpec((TC_BLK,), lambda i: (i,)),
        out_shape=jax.ShapeDtypeStruct((n,), jnp.float32),
    )(table_t, w_col)


def _sc_body(uid_hbm, iid_hbm, su_hbm, si_hbm, wb_hbm, out_hbm,
             idx_u, idx_i, g_u, g_i, out_v, wv, sem_u, sem_i):
    wid = lax.axis_index("s") * NC + lax.axis_index("c")
    base = wid * BPW

    pltpu.sync_copy(wb_hbm, wv)
    pltpu.sync_copy(uid_hbm.at[pl.ds(base, BPW)], idx_u)
    pltpu.sync_copy(iid_hbm.at[pl.ds(base, BPW)], idx_i)

    cu = pltpu.async_copy(su_hbm.at[idx_u], g_u, sem_u)
    ci = pltpu.async_copy(si_hbm.at[idx_i], g_i, sem_i)
    cu.wait()
    ci.wait()

    bias_vec = wv[pl.ds(0, L)]

    def group(g, carry):
        r0 = g * L
        out_v[pl.ds(r0, L)] = (
            g_u[pl.ds(r0, L)] + g_i[pl.ds(r0, L)] + bias_vec)
        return carry

    lax.fori_loop(0, BPW // L, group, 0)
    pltpu.sync_copy(out_v, out_hbm.at[pl.ds(base, BPW)])


@jax.jit
def _run(user_ids, item_ids, user_table, item_table, fc_w, fc_b):
    w = fc_w.reshape(-1)
    wu_col = w[:EMB].reshape(EMB, 1)
    wi_col = w[EMB:].reshape(EMB, 1)
    su = _col_dot(user_table.T, wu_col)
    si = _col_dot(item_table.T, wi_col)
    wb = jnp.broadcast_to(fc_b.reshape(-1), (L,))

    mesh = plsc.VectorSubcoreMesh(core_axis_name="c", subcore_axis_name="s")
    k = functools.partial(
        pl.kernel,
        mesh=mesh,
        compiler_params=pltpu.CompilerParams(
            needs_layout_passes=False, use_tc_tiling_on_sc=False),
        out_type=jax.ShapeDtypeStruct((BATCH,), jnp.float32),
        scratch_types=[
            pltpu.VMEM((BPW,), jnp.int32),
            pltpu.VMEM((BPW,), jnp.int32),
            pltpu.VMEM((BPW,), jnp.float32),
            pltpu.VMEM((BPW,), jnp.float32),
            pltpu.VMEM((BPW,), jnp.float32),
            pltpu.VMEM((L,), jnp.float32),
            pltpu.SemaphoreType.DMA,
            pltpu.SemaphoreType.DMA,
        ],
    )(_sc_body)
    return k(user_ids, item_ids, su, si, wb)


def kernel(user_ids, item_ids, user_table, item_table, fc_w, fc_b):
    return _run(user_ids.astype(jnp.int32), item_ids.astype(jnp.int32),
                user_table, item_table, fc_w, fc_b)


# speedup vs baseline: 7.8301x; 1.0457x over previous
import functools

import jax
import jax.numpy as jnp
from jax import lax
from jax.experimental import pallas as pl
from jax.experimental.pallas import tpu as pltpu
from jax.experimental.pallas import tpu_sc as plsc

BATCH = 16384
EMB = 32
L = 16

_info = plsc.get_sparse_core_info()
NC = _info.num_cores
NS = _info.num_subcores
NW = NC * NS
BPW = BATCH // NW

TC_BLK = 131072


def _col_dots(user_t, item_t, w_cols):
    nu = user_t.shape[1]
    ni = item_t.shape[1]

    def body(u_ref, i_ref, w_ref, su_ref, si_ref):
        w = w_ref[...]
        su_ref[...] = jax.lax.dot_general(
            w[:, 0:1], u_ref[...], (((0,), (0,)), ((), ())),
            preferred_element_type=jnp.float32).reshape(-1)

        @pl.when(pl.program_id(0) == 0)
        def _():
            si_ref[...] = jax.lax.dot_general(
                w[:, 1:2], i_ref[...], (((0,), (0,)), ((), ())),
                preferred_element_type=jnp.float32).reshape(-1)

    return pl.pallas_call(
        body,
        grid=(pl.cdiv(nu, TC_BLK),),
        in_specs=[
            pl.BlockSpec((EMB, TC_BLK), lambda i: (0, i)),
            pl.BlockSpec((EMB, ni), lambda i: (0, 0)),
            pl.BlockSpec((EMB, 2), lambda i: (0, 0)),
        ],
        out_specs=[
            pl.BlockSpec((TC_BLK,), lambda i: (i,)),
            pl.BlockSpec((ni,), lambda i: (0,)),
        ],
        out_shape=[
            jax.ShapeDtypeStruct((nu,), jnp.float32),
            jax.ShapeDtypeStruct((ni,), jnp.float32),
        ],
    )(user_t, item_t, w_cols)


def _sc_body(uid_hbm, iid_hbm, su_hbm, si_hbm, wb_hbm, out_hbm,
             idx_u, idx_i, g_u, g_i, out_v, wv, sem_u, sem_i):
    wid = lax.axis_index("s") * NC + lax.axis_index("c")
    base = wid * BPW

    pltpu.sync_copy(wb_hbm, wv)
    pltpu.sync_copy(uid_hbm.at[pl.ds(base, BPW)], idx_u)
    pltpu.sync_copy(iid_hbm.at[pl.ds(base, BPW)], idx_i)

    cu = pltpu.async_copy(su_hbm.at[idx_u], g_u, sem_u)
    ci = pltpu.async_copy(si_hbm.at[idx_i], g_i, sem_i)
    cu.wait()
    ci.wait()

    bias_vec = wv[pl.ds(0, L)]

    def group(g, carry):
        r0 = g * L
        out_v[pl.ds(r0, L)] = (
            g_u[pl.ds(r0, L)] + g_i[pl.ds(r0, L)] + bias_vec)
        return carry

    lax.fori_loop(0, BPW // L, group, 0)
    pltpu.sync_copy(out_v, out_hbm.at[pl.ds(base, BPW)])


@jax.jit
def _run(user_ids, item_ids, user_table, item_table, fc_w, fc_b):
    w_cols = fc_w.reshape(2, EMB).T
    su, si = _col_dots(user_table.T, item_table.T, w_cols)
    wb = jnp.broadcast_to(fc_b.reshape(-1), (L,))

    mesh = plsc.VectorSubcoreMesh(core_axis_name="c", subcore_axis_name="s")
    k = functools.partial(
        pl.kernel,
        mesh=mesh,
        compiler_params=pltpu.CompilerParams(
            needs_layout_passes=False, use_tc_tiling_on_sc=False),
        out_type=jax.ShapeDtypeStruct((BATCH,), jnp.float32),
        scratch_types=[
            pltpu.VMEM((BPW,), jnp.int32),
            pltpu.VMEM((BPW,), jnp.int32),
            pltpu.VMEM((BPW,), jnp.float32),
            pltpu.VMEM((BPW,), jnp.float32),
            pltpu.VMEM((BPW,), jnp.float32),
            pltpu.VMEM((L,), jnp.float32),
            pltpu.SemaphoreType.DMA,
            pltpu.SemaphoreType.DMA,
        ],
    )(_sc_body)
    return k(user_ids, item_ids, su, si, wb)


def kernel(user_ids, item_ids, user_table, item_table, fc_w, fc_b):
    return _run(user_ids.astype(jnp.int32), item_ids.astype(jnp.int32),
                user_table, item_table, fc_w, fc_b)


# speedup vs baseline: 8.1482x; 1.0406x over previous
import functools

import jax
import jax.numpy as jnp
from jax import lax
from jax.experimental import pallas as pl
from jax.experimental.pallas import tpu as pltpu
from jax.experimental.pallas import tpu_sc as plsc

BATCH = 16384
EMB = 32
L = 16

_info = plsc.get_sparse_core_info()
NC = _info.num_cores
NS = _info.num_subcores
NW = NC * NS
BPW = BATCH // NW

TC_BLK = 131072


def _col_dots(user_t, item_t, w_row):
    nu = user_t.shape[1]
    ni = item_t.shape[1]

    def body(u_ref, i_ref, w_ref, su_ref, si_ref):
        w = w_ref[...]
        su_ref[...] = jax.lax.dot_general(
            w[0:1, 0:EMB], u_ref[...], (((1,), (0,)), ((), ())),
            preferred_element_type=jnp.float32).reshape(-1)

        @pl.when(pl.program_id(0) == 0)
        def _():
            si_ref[...] = jax.lax.dot_general(
                w[0:1, EMB:2 * EMB], i_ref[...], (((1,), (0,)), ((), ())),
                preferred_element_type=jnp.float32).reshape(-1)

    return pl.pallas_call(
        body,
        grid=(pl.cdiv(nu, TC_BLK),),
        in_specs=[
            pl.BlockSpec((EMB, TC_BLK), lambda i: (0, i)),
            pl.BlockSpec((EMB, ni), lambda i: (0, 0)),
            pl.BlockSpec((1, 2 * EMB), lambda i: (0, 0)),
        ],
        out_specs=[
            pl.BlockSpec((TC_BLK,), lambda i: (i,)),
            pl.BlockSpec((ni,), lambda i: (0,)),
        ],
        out_shape=[
            jax.ShapeDtypeStruct((nu,), jnp.float32),
            jax.ShapeDtypeStruct((ni,), jnp.float32),
        ],
    )(user_t, item_t, w_row)


def _sc_body(uid_hbm, iid_hbm, su_hbm, si_hbm, wb_hbm, out_hbm,
             idx_u, idx_i, g_u, g_i, out_v, wv, sem_u, sem_i):
    wid = lax.axis_index("s") * NC + lax.axis_index("c")
    base = wid * BPW

    pltpu.sync_copy(wb_hbm, wv)
    pltpu.sync_copy(uid_hbm.at[pl.ds(base, BPW)], idx_u)
    pltpu.sync_copy(iid_hbm.at[pl.ds(base, BPW)], idx_i)

    cu = pltpu.async_copy(su_hbm.at[idx_u], g_u, sem_u)
    ci = pltpu.async_copy(si_hbm.at[idx_i], g_i, sem_i)
    cu.wait()
    ci.wait()

    bias_vec = wv[pl.ds(0, L)]

    def group(g, carry):
        r0 = g * L
        out_v[pl.ds(r0, L)] = (
            g_u[pl.ds(r0, L)] + g_i[pl.ds(r0, L)] + bias_vec)
        return carry

    lax.fori_loop(0, BPW // L, group, 0)
    pltpu.sync_copy(out_v, out_hbm.at[pl.ds(base, BPW)])


@jax.jit
def _run(user_ids, item_ids, user_table, item_table, fc_w, fc_b):
    su, si = _col_dots(user_table.T, item_table.T, fc_w)
    wb = jnp.broadcast_to(fc_b.reshape(-1), (L,))

    mesh = plsc.VectorSubcoreMesh(core_axis_name="c", subcore_axis_name="s")
    k = functools.partial(
        pl.kernel,
        mesh=mesh,
        compiler_params=pltpu.CompilerParams(
            needs_layout_passes=False, use_tc_tiling_on_sc=False),
        out_type=jax.ShapeDtypeStruct((BATCH,), jnp.float32),
        scratch_types=[
            pltpu.VMEM((BPW,), jnp.int32),
            pltpu.VMEM((BPW,), jnp.int32),
            pltpu.VMEM((BPW,), jnp.float32),
            pltpu.VMEM((BPW,), jnp.float32),
            pltpu.VMEM((BPW,), jnp.float32),
            pltpu.VMEM((L,), jnp.float32),
            pltpu.SemaphoreType.DMA,
            pltpu.SemaphoreType.DMA,
        ],
    )(_sc_body)
    return k(user_ids, item_ids, su, si, wb)


def kernel(user_ids, item_ids, user_table, item_table, fc_w, fc_b):
    return _run(user_ids.astype(jnp.int32), item_ids.astype(jnp.int32),
                user_table, item_table, fc_w, fc_b)
